# baseline (speedup 1.0000x reference)
import functools

import jax
import jax.numpy as jnp
from jax import lax
from jax.experimental import pallas as pl
from jax.experimental.pallas import tpu as pltpu
from jax.experimental.pallas import tpu_sc as plsc

_N = 100000
_E = 6400000

_NCORES = 2
_NTILES = 16
_NWORK = _NCORES * _NTILES
_NP = 100352
_CH = _E // 128
_WC = 22
_NWIN = 71
_CPW_LO = 1562
_CPW_HI = 1563


def _sc_degree_partials(ei3, edge_weight):
    mesh = plsc.VectorSubcoreMesh(core_axis_name="c", subcore_axis_name="s")

    @functools.partial(
        pl.kernel,
        out_type=(
            jax.ShapeDtypeStruct((_NWORK, _NP), jnp.float32),
            jax.ShapeDtypeStruct((_NWORK, _NP), jnp.float32),
        ),
        mesh=mesh,
        compiler_params=pltpu.CompilerParams(needs_layout_passes=False),
        scratch_types=(
            pltpu.VMEM((_WC, 2, 128), jnp.int32),
            pltpu.VMEM((_WC, 2, 128), jnp.int32),
            pltpu.VMEM((_WC * 128,), jnp.float32),
            pltpu.VMEM((_WC * 128,), jnp.float32),
            pltpu.VMEM((_NP,), jnp.float32),
            pltpu.SemaphoreType.DMA((2,)),
        ),
    )
    def k(ei_hbm, w_hbm, dout_hbm, din_hbm,
          idx0, idx1, wb0, wb1, acc, lsem):
        c = lax.axis_index("c")
        s = lax.axis_index("s")
        wid = c * _NTILES + s
        zero16 = jnp.zeros((16,), jnp.float32)
        chunk0 = jnp.where(wid < 16, wid * _CPW_HI, wid * _CPW_LO + 16)

        def start_load(i, r, ib, vb, sem):
            ci = chunk0 + i * _WC
            pltpu.async_copy(ei_hbm.at[pl.ds(ci, _WC)], ib, sem)
            pltpu.async_copy(w_hbm.at[pl.ds(ci * 128, _WC * 128)], vb, sem)

        def wait_load(r, ib, vb, sem):
            pltpu.make_async_copy(ei_hbm.at[pl.ds(0, _WC)], ib, sem).wait()
            pltpu.make_async_copy(w_hbm.at[pl.ds(0, _WC * 128)], vb,
                                  sem).wait()

        def scatter_window(r, ib, vb):
            @plsc.parallel_loop(0, _WC, 1, unroll=2)
            def _(kk):
                for u in range(8):
                    iv = ib[kk, r, pl.ds(u * 16, 16)]
                    vv = vb[pl.ds(kk * 128 + u * 16, 16)]
                    plsc.addupdate_scatter(acc, [iv], vv)

        def one_pass(r, out_hbm):
            @plsc.parallel_loop(0, _NP // 16, 1, unroll=8)
            def _(i):
                acc[pl.ds(i * 16, 16)] = zero16

            start_load(0, r, idx0, wb0, lsem.at[0])
            start_load(1, r, idx1, wb1, lsem.at[1])

            def win(k2, carry):
                i = k2 * 2
                wait_load(r, idx0, wb0, lsem.at[0])
                scatter_window(r, idx0, wb0)
                start_load(i + 2, r, idx0, wb0, lsem.at[0])

                wait_load(r, idx1, wb1, lsem.at[1])
                scatter_window(r, idx1, wb1)

                @pl.when(i + 3 < _NWIN)
                def _():
                    start_load(i + 3, r, idx1, wb1, lsem.at[1])
                return carry
            lax.fori_loop(0, _NWIN // 2, win, 0)

            wait_load(r, idx0, wb0, lsem.at[0])
            scatter_window(r, idx0, wb0)

            @pl.when(wid < 16)
            def _():
                tc = chunk0 + _CPW_LO
                pltpu.sync_copy(ei_hbm.at[pl.ds(tc, 1)],
                                idx0.at[pl.ds(0, 1)])
                pltpu.sync_copy(w_hbm.at[pl.ds(tc * 128, 128)],
                                wb0.at[pl.ds(0, 128)])
                for u in range(8):
                    plsc.addupdate_scatter(acc, [idx0[0, r, pl.ds(u * 16, 16)]],
                                           wb0[pl.ds(u * 16, 16)])

            pltpu.sync_copy(acc, out_hbm.at[wid])

        one_pass(0, dout_hbm)
        one_pass(1, din_hbm)

    return k(ei3, edge_weight)


_ROWS = _E // 128
_BR = 2000
_GB = _ROWS // _BR


def _tc_sums(w2d):
    def body(w_ref, s_ref, wl_ref):
        i = pl.program_id(0)
        w = w_ref[...]

        @pl.when(i == 0)
        def _():
            s_ref[0, 0] = 0.0
            wl_ref[0, 0] = 0.0

        s_ref[0, 0] += jnp.sum(w)
        wl_ref[0, 0] += jnp.sum(w * jnp.log(w))

    return pl.pallas_call(
        body,
        grid=(_GB,),
        in_specs=[pl.BlockSpec((_BR, 128), lambda i: (i, 0))],
        out_specs=[
            pl.BlockSpec(memory_space=pltpu.SMEM),
            pl.BlockSpec(memory_space=pltpu.SMEM),
        ],
        out_shape=(
            jax.ShapeDtypeStruct((1, 1), jnp.float32),
            jax.ShapeDtypeStruct((1, 1), jnp.float32),
        ),
    )(w2d)


def _tc_puv(w2d, s):
    def body(w_ref, s_ref, puv_ref):
        puv_ref[...] = w_ref[...] * (1.0 / s_ref[0, 0])

    return pl.pallas_call(
        body,
        grid=(_GB,),
        in_specs=[
            pl.BlockSpec((_BR, 128), lambda i: (i, 0)),
            pl.BlockSpec(memory_space=pltpu.SMEM),
        ],
        out_specs=pl.BlockSpec((_BR, 128), lambda i: (i, 0)),
        out_shape=jax.ShapeDtypeStruct((_ROWS, 128), jnp.float32),
    )(w2d, s)


def _tc_epilogue(dout2, din2, s, swl):
    def body(dout_ref, din_ref, s_ref, swl_ref, pout_ref, pin_ref, mi_ref):
        dout = dout_ref[pl.ds(0, 784), :]
        din = din_ref[pl.ds(0, 784), :]
        for j in range(1, _NWORK):
            dout = dout + dout_ref[pl.ds(j * 784, 784), :]
            din = din + din_ref[pl.ds(j * 784, 784), :]
        s_out = jnp.sum(dout)
        s_in = jnp.sum(din)
        pout = dout / s_out
        pin = din / s_in
        pout_ref[...] = pout
        pin_ref[...] = pin
        r = jnp.sum(
            jnp.where(dout > 0, dout * jnp.log(pin), 0.0)
            + jnp.where(din > 0, din * jnp.log(pout), 0.0)
        )
        sval = s_ref[0, 0]
        mi_ref[0, 0] = swl_ref[0, 0] / sval - jnp.log(sval) - r / sval

    return pl.pallas_call(
        body,
        in_specs=[
            pl.BlockSpec(memory_space=pltpu.VMEM),
            pl.BlockSpec(memory_space=pltpu.VMEM),
            pl.BlockSpec(memory_space=pltpu.SMEM),
            pl.BlockSpec(memory_space=pltpu.SMEM),
        ],
        out_specs=(
            pl.BlockSpec(memory_space=pltpu.VMEM),
            pl.BlockSpec(memory_space=pltpu.VMEM),
            pl.BlockSpec(memory_space=pltpu.SMEM),
        ),
        out_shape=(
            jax.ShapeDtypeStruct((784, 128), jnp.float32),
            jax.ShapeDtypeStruct((784, 128), jnp.float32),
            jax.ShapeDtypeStruct((1, 1), jnp.float32),
        ),
    )(dout2, din2, s, swl)


def kernel(edge_index, edge_weight):
    w2d = edge_weight.reshape(_ROWS, 128)
    ei3 = jnp.transpose(edge_index.reshape(2, _CH, 128), (1, 0, 2))
    dout_p, din_p = _sc_degree_partials(ei3, edge_weight)
    s, swl = _tc_sums(w2d)
    puv2d = _tc_puv(w2d, s)
    pout_pad, pin_pad, mi = _tc_epilogue(
        dout_p.reshape(_NWORK * 784, 128), din_p.reshape(_NWORK * 784, 128),
        s, swl,
    )

    node_ids = jnp.arange(_N, dtype=jnp.int32)
    p_uv = puv2d.reshape(_E)
    p_u_out = pout_pad.reshape(_NP)[:_N]
    p_u_in = pin_pad.reshape(_NP)[:_N]
    return (node_ids, p_uv, p_u_out, p_u_in, mi.reshape(()))


# speedup vs baseline: 1.2433x; 1.2433x over previous
import functools

import jax
import jax.numpy as jnp
from jax import lax
from jax.experimental import pallas as pl
from jax.experimental.pallas import tpu as pltpu
from jax.experimental.pallas import tpu_sc as plsc

_N = 100000
_E = 6400000

_NCORES = 2
_NTILES = 16
_NWORK = _NCORES * _NTILES
_NP = 100352
_CH = _E // 128
_WC = 50
_TOTWIN = _CH // _WC


def _sc_degree_partials(ei3, edge_weight):
    mesh = plsc.VectorSubcoreMesh(core_axis_name="c", subcore_axis_name="s")

    @functools.partial(
        pl.kernel,
        out_type=(
            jax.ShapeDtypeStruct((_NWORK, _NP), jnp.float32),
            jax.ShapeDtypeStruct((_NWORK, _NP), jnp.float32),
        ),
        mesh=mesh,
        compiler_params=pltpu.CompilerParams(needs_layout_passes=False),
        scratch_types=(
            pltpu.VMEM((_WC, 128), jnp.int32),
            pltpu.VMEM((_WC, 128), jnp.int32),
            pltpu.VMEM((_WC * 128,), jnp.float32),
            pltpu.VMEM((_WC * 128,), jnp.float32),
            pltpu.VMEM((_NP,), jnp.float32),
            pltpu.SemaphoreType.DMA((2,)),
        ),
    )
    def k(ei_hbm, w_hbm, dout_hbm, din_hbm,
          idx0, idx1, wb0, wb1, acc, lsem):
        c = lax.axis_index("c")
        s = lax.axis_index("s")
        wid = c * _NTILES + s
        zero16 = jnp.zeros((16,), jnp.float32)
        win0 = wid * 31 + jnp.minimum(wid, 8)
        nwin = jnp.where(wid < 8, 32, 31)
        chunk0 = win0 * _WC

        def start_load(i, r, ib, vb, sem):
            ci = chunk0 + i * _WC
            pltpu.async_copy(ei_hbm.at[pl.ds(ci, _WC), r], ib, sem)
            pltpu.async_copy(w_hbm.at[pl.ds(ci * 128, _WC * 128)], vb, sem)

        def wait_load(r, ib, vb, sem):
            pltpu.make_async_copy(ei_hbm.at[pl.ds(0, _WC), r], ib, sem).wait()
            pltpu.make_async_copy(w_hbm.at[pl.ds(0, _WC * 128)], vb,
                                  sem).wait()

        def scatter_window(ib, vb):
            @plsc.parallel_loop(0, _WC, 1, unroll=2)
            def _(kk):
                for u in range(8):
                    iv = ib[kk, pl.ds(u * 16, 16)]
                    vv = vb[pl.ds(kk * 128 + u * 16, 16)]
                    plsc.addupdate_scatter(acc, [iv], vv)

        def one_pass(r, out_hbm):
            @plsc.parallel_loop(0, _NP // 16, 1, unroll=8)
            def _(i):
                acc[pl.ds(i * 16, 16)] = zero16

            start_load(0, r, idx0, wb0, lsem.at[0])
            start_load(1, r, idx1, wb1, lsem.at[1])

            def win(k2, carry):
                i = k2 * 2
                wait_load(r, idx0, wb0, lsem.at[0])
                scatter_window(idx0, wb0)

                @pl.when(i + 2 < nwin)
                def _():
                    start_load(i + 2, r, idx0, wb0, lsem.at[0])

                wait_load(r, idx1, wb1, lsem.at[1])
                scatter_window(idx1, wb1)

                @pl.when(i + 3 < nwin)
                def _():
                    start_load(i + 3, r, idx1, wb1, lsem.at[1])
                return carry
            lax.fori_loop(0, nwin // 2, win, 0)

            @pl.when(nwin % 2 == 1)
            def _():
                wait_load(r, idx0, wb0, lsem.at[0])
                scatter_window(idx0, wb0)

            pltpu.sync_copy(acc, out_hbm.at[wid])

        one_pass(0, dout_hbm)
        one_pass(1, din_hbm)

    return k(ei3, edge_weight)


_ROWS = _E // 128
_BR = 2000
_GB = _ROWS // _BR


def _tc_sums(w2d):
    def body(w_ref, s_ref, wl_ref):
        i = pl.program_id(0)
        w = w_ref[...]

        @pl.when(i == 0)
        def _():
            s_ref[0, 0] = 0.0
            wl_ref[0, 0] = 0.0

        s_ref[0, 0] += jnp.sum(w)
        wl_ref[0, 0] += jnp.sum(w * jnp.log(w))

    return pl.pallas_call(
        body,
        grid=(_GB,),
        in_specs=[pl.BlockSpec((_BR, 128), lambda i: (i, 0))],
        out_specs=[
            pl.BlockSpec(memory_space=pltpu.SMEM),
            pl.BlockSpec(memory_space=pltpu.SMEM),
        ],
        out_shape=(
            jax.ShapeDtypeStruct((1, 1), jnp.float32),
            jax.ShapeDtypeStruct((1, 1), jnp.float32),
        ),
    )(w2d)


def _tc_puv(w2d, s):
    def body(w_ref, s_ref, puv_ref):
        puv_ref[...] = w_ref[...] * (1.0 / s_ref[0, 0])

    return pl.pallas_call(
        body,
        grid=(_GB,),
        in_specs=[
            pl.BlockSpec((_BR, 128), lambda i: (i, 0)),
            pl.BlockSpec(memory_space=pltpu.SMEM),
        ],
        out_specs=pl.BlockSpec((_BR, 128), lambda i: (i, 0)),
        out_shape=jax.ShapeDtypeStruct((_ROWS, 128), jnp.float32),
    )(w2d, s)


def _tc_epilogue(dout2, din2, s, swl):
    def body(dout_ref, din_ref, s_ref, swl_ref, pout_ref, pin_ref, mi_ref):
        dout = dout_ref[pl.ds(0, 784), :]
        din = din_ref[pl.ds(0, 784), :]
        for j in range(1, _NWORK):
            dout = dout + dout_ref[pl.ds(j * 784, 784), :]
            din = din + din_ref[pl.ds(j * 784, 784), :]
        s_out = jnp.sum(dout)
        s_in = jnp.sum(din)
        pout = dout / s_out
        pin = din / s_in
        pout_ref[...] = pout
        pin_ref[...] = pin
        r = jnp.sum(
            jnp.where(dout > 0, dout * jnp.log(pin), 0.0)
            + jnp.where(din > 0, din * jnp.log(pout), 0.0)
        )
        sval = s_ref[0, 0]
        mi_ref[0, 0] = swl_ref[0, 0] / sval - jnp.log(sval) - r / sval

    return pl.pallas_call(
        body,
        in_specs=[
            pl.BlockSpec(memory_space=pltpu.VMEM),
            pl.BlockSpec(memory_space=pltpu.VMEM),
            pl.BlockSpec(memory_space=pltpu.SMEM),
            pl.BlockSpec(memory_space=pltpu.SMEM),
        ],
        out_specs=(
            pl.BlockSpec(memory_space=pltpu.VMEM),
            pl.BlockSpec(memory_space=pltpu.VMEM),
            pl.BlockSpec(memory_space=pltpu.SMEM),
        ),
        out_shape=(
            jax.ShapeDtypeStruct((784, 128), jnp.float32),
            jax.ShapeDtypeStruct((784, 128), jnp.float32),
            jax.ShapeDtypeStruct((1, 1), jnp.float32),
        ),
    )(dout2, din2, s, swl)


def kernel(edge_index, edge_weight):
    w2d = edge_weight.reshape(_ROWS, 128)
    ei3 = jnp.transpose(edge_index.reshape(2, _CH, 128), (1, 0, 2))
    dout_p, din_p = _sc_degree_partials(ei3, edge_weight)
    s, swl = _tc_sums(w2d)
    puv2d = _tc_puv(w2d, s)
    pout_pad, pin_pad, mi = _tc_epilogue(
        dout_p.reshape(_NWORK * 784, 128), din_p.reshape(_NWORK * 784, 128),
        s, swl,
    )

    node_ids = jnp.arange(_N, dtype=jnp.int32)
    p_uv = puv2d.reshape(_E)
    p_u_out = pout_pad.reshape(_NP)[:_N]
    p_u_in = pin_pad.reshape(_NP)[:_N]
    return (node_ids, p_uv, p_u_out, p_u_in, mi.reshape(()))
